# baseline (speedup 1.0000x reference)
import jax
import jax.numpy as jnp
from jax.experimental import pallas as pl

_EPS = 1e-5


def _fold(cm, groups, c):
    s = cm[:, 0:c]
    for i in range(1, groups):
        s = s + cm[:, i * c:(i + 1) * c]
    return jnp.concatenate([s / groups] * groups, axis=1)


def _packed_bn_relu(y, groups, c, g, b):
    cm = _fold(jnp.mean(y, axis=0, keepdims=True), groups, c)
    yc = y - cm
    cv = _fold(jnp.mean(yc * yc, axis=0, keepdims=True), groups, c)
    return jnp.maximum(g * yc / jnp.sqrt(cv + _EPS) + b, 0.0)


def _mm_bn8_body(x_ref, w_ref, b_ref, g_ref, bb_ref, o_ref):
    y = jnp.dot(x_ref[...], w_ref[...], preferred_element_type=jnp.float32)
    o_ref[...] = _packed_bn_relu(y + b_ref[...], 8, 16, g_ref[...], bb_ref[...])


def _cls4_body(x_ref, w_ref, b_ref, g_ref, bb_ref, o_ref):
    ys = [
        jnp.dot(x_ref[i], w_ref[i], preferred_element_type=jnp.float32)
        for i in range(4)
    ]
    y = jnp.stack(ys, axis=0) + b_ref[...]
    cm = jnp.mean(y, axis=(0, 1), keepdims=True)
    cm = _fold(cm[0], 4, 16)[None]
    yc = y - cm
    cv = jnp.mean(yc * yc, axis=(0, 1), keepdims=True)
    cv = _fold(cv[0], 4, 16)[None]
    o_ref[...] = jnp.maximum(
        g_ref[...] * yc / jnp.sqrt(cv + _EPS) + bb_ref[...], 0.0)


def _mid_body(x_ref, w3_ref, b3_ref, g3_ref, bb3_ref,
              w4_ref, b4_ref, g4_ref, bb4_ref,
              embt_ref, emb_ref,
              wd_ref, bd_ref, gd_ref, bbd_ref,
              ze_ref, lat_ref, zq_ref, hd_ref):
    h3 = jnp.dot(x_ref[...], w3_ref[...], preferred_element_type=jnp.float32)
    h3 = _packed_bn_relu(h3 + b3_ref[...], 1, 16, g3_ref[...], bb3_ref[...])
    z = jnp.dot(h3, w4_ref[...], preferred_element_type=jnp.float32)
    z = _packed_bn_relu(z + b4_ref[...], 1, 32, g4_ref[...], bb4_ref[...])
    ze_ref[...] = z

    m = z.shape[0]
    k = emb_ref.shape[0]
    embt = embt_ref[...]
    acc = jnp.zeros((m, k), jnp.float32)
    for c in range(32):
        d = z[:, c:c + 1] - embt[c:c + 1, :]
        acc = acc + d * d
    dmin = jnp.min(acc, axis=1, keepdims=True)
    iota = jax.lax.broadcasted_iota(jnp.int32, (m, k), 1)
    lat = jnp.min(jnp.where(acc == dmin, iota, k), axis=1, keepdims=True)
    lat_ref[...] = lat

    onehot = (iota == lat).astype(jnp.float32)
    zq = jnp.dot(onehot, emb_ref[...], preferred_element_type=jnp.float32)
    zq_ref[...] = zq
    hd = jnp.dot(zq, wd_ref[...], preferred_element_type=jnp.float32)
    hd_ref[...] = _packed_bn_relu(hd + bd_ref[...], 1, 16,
                                  gd_ref[...], bbd_ref[...])


def _dec3_body(x_ref, w_ref, b_ref, g_ref, bb_ref, mask_ref, o_ref):
    y = jnp.dot(x_ref[...], w_ref[...], preferred_element_type=jnp.float32)
    y = y + b_ref[...]
    mask = mask_ref[...]
    cnt = 51984.0
    cm = _fold(jnp.sum(y * mask, axis=0, keepdims=True), 32, 16) / (cnt / 32.0)
    yc = y - cm
    cv = _fold(jnp.sum(yc * yc * mask, axis=0, keepdims=True), 32, 16) / (cnt / 32.0)
    o_ref[...] = jnp.maximum(
        g_ref[...] * yc / jnp.sqrt(cv + _EPS) + bb_ref[...], 0.0)


def _dec4_body(x_ref, w_ref, b_ref, g_ref, bb_ref, wo_ref, bo_ref, o_ref):
    y = jnp.dot(x_ref[...], w_ref[...], preferred_element_type=jnp.float32)
    yr = _packed_bn_relu(y + b_ref[...], 32, 16, g_ref[...], bb_ref[...])
    o_ref[...] = (
        jnp.dot(yr, wo_ref[...], preferred_element_type=jnp.float32)
        + bo_ref[...])


def _call(body, out_shapes, *args):
    return pl.pallas_call(body, out_shape=out_shapes)(*args)


def _row(a, reps=1):
    a = jnp.tile(a, reps) if reps > 1 else a
    return a.reshape(1, -1)


def _im2col_s2(h, k, pad):
    n, hh, ww, c = h.shape
    hp = jnp.pad(h, ((0, 0), (pad, pad), (pad, pad), (0, 0)))
    ho = (hh + 2 * pad - k) // 2 + 1
    cols = []
    for kh in range(k):
        for kw in range(k):
            cols.append(hp[:, kh:kh + 2 * ho - 1:2, kw:kw + 2 * ho - 1:2, :])
    x = jnp.stack(cols, axis=3)
    return x.reshape(n * ho * ho, k * k * c), ho


def _wmat(w):
    co, ci, kh, kw = w.shape
    return jnp.transpose(w, (2, 3, 1, 0)).reshape(kh * kw * ci, co)


def _blk(w, reps):
    return jnp.kron(jnp.eye(reps, dtype=w.dtype), w)


def kernel(x, params):
    p = params
    n = x.shape[0]
    xh = jnp.transpose(x, (0, 2, 3, 1))

    x1, h1o = _im2col_s2(xh, 4, 1)
    m1 = x1.shape[0]
    f1 = _call(
        _mm_bn8_body,
        jax.ShapeDtypeStruct((m1 // 8, 128), jnp.float32),
        x1.reshape(m1 // 8, 8 * 48), _blk(_wmat(p['ew1']), 8),
        _row(p['eb1'], 8), _row(p['eg1'], 8), _row(p['ebb1'], 8))
    h1 = f1.reshape(n, h1o, h1o, 16)

    x2, h2o = _im2col_s2(h1, 4, 1)
    m2 = x2.shape[0]
    f2 = _call(
        _mm_bn8_body,
        jax.ShapeDtypeStruct((m2 // 8, 128), jnp.float32),
        x2.reshape(m2 // 8, 8 * 256), _blk(_wmat(p['ew2']), 8),
        _row(p['eb2'], 8), _row(p['eg2'], 8), _row(p['ebb2'], 8))
    h2 = f2.reshape(n, h2o, h2o, 16)

    x3, h3o = _im2col_s2(h2, 2, 1)
    m3 = x3.shape[0]
    w4 = jnp.transpose(p['ew4'][:, :, 0, 0], (1, 0))
    wd1 = p['dw1'][:, :, 0, 0]
    ze_f, lat_f, zq_f, hd1_f = _call(
        _mid_body,
        (jax.ShapeDtypeStruct((m3, 32), jnp.float32),
         jax.ShapeDtypeStruct((m3, 1), jnp.int32),
         jax.ShapeDtypeStruct((m3, 32), jnp.float32),
         jax.ShapeDtypeStruct((m3, 16), jnp.float32)),
        x3, _wmat(p['ew3']), _row(p['eb3']), _row(p['eg3']), _row(p['ebb3']),
        w4, _row(p['eb4']), _row(p['eg4']), _row(p['ebb4']),
        jnp.transpose(p['emb'], (1, 0)), p['emb'],
        wd1, _row(p['db1']), _row(p['dg1']), _row(p['dbb1']))

    z_e_x = jnp.transpose(ze_f.reshape(n, h3o, h3o, 32), (0, 3, 1, 2))
    z_q_x = jnp.transpose(zq_f.reshape(n, h3o, h3o, 32), (0, 3, 1, 2))
    latents = lat_f.reshape(n, h3o, h3o)
    hd1 = hd1_f.reshape(n, h3o, h3o, 16)

    hp = jnp.pad(hd1, ((0, 0), (1, 1), (1, 1), (0, 0)))
    xs, ws = [], []
    w2 = p['dw2']
    for a in range(2):
        for b in range(2):
            taps = []
            wt = []
            for rh in range(2):
                for rw in range(2):
                    taps.append(hp[:, a + rh:a + rh + h3o,
                                   b + rw:b + rw + h3o, :])
                    wt.append(w2[:, :, 3 - a - 2 * rh, 3 - b - 2 * rw])
            xc = jnp.stack(taps, axis=3).reshape(m3 // 4, 4 * 64)
            wc = jnp.concatenate(wt, axis=0)
            xs.append(xc)
            ws.append(_blk(wc, 4))
    xcls = jnp.stack(xs, axis=0)
    wcls = jnp.stack(ws, axis=0)
    d2c = _call(
        _cls4_body,
        jax.ShapeDtypeStruct((4, m3 // 4, 64), jnp.float32),
        xcls, wcls, _row(p['db2'], 4), _row(p['dg2'], 4), _row(p['dbb2'], 4))
    d2 = d2c.reshape(2, 2, n, h3o, h3o, 16)
    d2 = jnp.transpose(d2, (2, 3, 0, 4, 1, 5)).reshape(n, 2 * h3o, 2 * h3o, 16)

    s2b = 2 * h3o
    s3 = s2b - 1
    m5 = n * s2b * s2b
    w3d = p['dw3']
    wall = jnp.concatenate(
        [w3d[:, :, 1 - a, 1 - b] for a in range(2) for b in range(2)], axis=1)
    pos = jnp.arange(m5)
    hh = (pos % (s2b * s2b)) // s2b
    wwp = pos % s2b
    vm = []
    for a in range(2):
        for b in range(2):
            vm.append(((hh >= a) & (hh <= s3 - 1 + a)
                       & (wwp >= b) & (wwp <= s3 - 1 + b)))
    mask = jnp.stack(vm, axis=1).astype(jnp.float32)
    mask = jnp.repeat(mask, 16, axis=1).reshape(m5 // 8, 512)
    d3c = _call(
        _dec3_body,
        jax.ShapeDtypeStruct((m5 // 8, 512), jnp.float32),
        d2.reshape(m5 // 8, 128), _blk(wall, 8),
        _row(p['db3'], 32), _row(p['dg3'], 32), _row(p['dbb3'], 32), mask)
    d3full = d3c.reshape(n, s2b, s2b, 4, 16)
    cls = []
    for a in range(2):
        for b in range(2):
            cls.append(d3full[:, a:a + s3, b:b + s3, 2 * a + b, :])
    d3 = jnp.stack(cls, axis=0).reshape(2, 2, n, s3, s3, 16)
    d3 = jnp.transpose(d3, (2, 3, 0, 4, 1, 5)).reshape(n, 2 * s3, 2 * s3, 16)

    s4 = 2 * s3
    m6 = n * s4 * s4
    w4d = p['dw4']
    wcat = jnp.concatenate(
        [w4d[:, :, a, b] for a in range(2) for b in range(2)], axis=1)
    wo = p['ow'][:, :, 0, 0]
    zeros = jnp.zeros_like(wo)
    wblk = jnp.concatenate([
        jnp.concatenate([wo if i == j else zeros for j in range(4)], axis=1)
        for i in range(4)], axis=0)
    out6 = _call(
        _dec4_body,
        jax.ShapeDtypeStruct((m6 // 8, 96), jnp.float32),
        d3.reshape(m6 // 8, 128), _blk(wcat, 8),
        _row(p['db4'], 32), _row(p['dg4'], 32), _row(p['dbb4'], 32),
        _blk(wblk, 8), _row(p['ob'], 32))
    xt = out6.reshape(n, s4, s4, 2, 2, 3)
    xt = jnp.transpose(xt, (0, 1, 3, 2, 4, 5)).reshape(n, 2 * s4, 2 * s4, 3)
    x_tilde = jnp.transpose(xt, (0, 3, 1, 2))

    return x_tilde, z_e_x, z_q_x, latents


# speedup vs baseline: 1.6756x; 1.6756x over previous
import jax
import jax.numpy as jnp
from jax.experimental import pallas as pl

_EPS = 1e-5


def _fold(cm, groups, c):
    s = cm[:, 0:c]
    for i in range(1, groups):
        s = s + cm[:, i * c:(i + 1) * c]
    return jnp.concatenate([s / groups] * groups, axis=1)


def _packed_bn_relu(y, groups, c, g, b):
    cm = _fold(jnp.mean(y, axis=0, keepdims=True), groups, c)
    yc = y - cm
    cv = _fold(jnp.mean(yc * yc, axis=0, keepdims=True), groups, c)
    return jnp.maximum(g * yc / jnp.sqrt(cv + _EPS) + b, 0.0)


def _mm_bn8_body(x_ref, w_ref, b_ref, g_ref, bb_ref, o_ref):
    y = jnp.dot(x_ref[...], w_ref[...], preferred_element_type=jnp.float32)
    o_ref[...] = _packed_bn_relu(y + b_ref[...], 8, 16, g_ref[...], bb_ref[...])


def _cls4_body(x_ref, w_ref, b_ref, g_ref, bb_ref, o_ref):
    ys = [
        jnp.dot(x_ref[i], w_ref[i], preferred_element_type=jnp.float32)
        for i in range(4)
    ]
    y = jnp.stack(ys, axis=0) + b_ref[...]
    cm = jnp.mean(y, axis=(0, 1), keepdims=True)
    cm = _fold(cm[0], 4, 16)[None]
    yc = y - cm
    cv = jnp.mean(yc * yc, axis=(0, 1), keepdims=True)
    cv = _fold(cv[0], 4, 16)[None]
    o_ref[...] = jnp.maximum(
        g_ref[...] * yc / jnp.sqrt(cv + _EPS) + bb_ref[...], 0.0)


def _mid_body(x_ref, w3_ref, b3_ref, g3_ref, bb3_ref,
              w4_ref, b4_ref, g4_ref, bb4_ref,
              embt_ref, emb_ref,
              wd_ref, bd_ref, gd_ref, bbd_ref,
              ze_ref, lat_ref, zq_ref, hd_ref):
    h3 = jnp.dot(x_ref[...], w3_ref[...], preferred_element_type=jnp.float32)
    h3 = _packed_bn_relu(h3 + b3_ref[...], 1, 16, g3_ref[...], bb3_ref[...])
    z = jnp.dot(h3, w4_ref[...], preferred_element_type=jnp.float32)
    z = _packed_bn_relu(z + b4_ref[...], 1, 32, g4_ref[...], bb4_ref[...])
    ze_ref[...] = z

    m = z.shape[0]
    k = emb_ref.shape[0]
    embt = embt_ref[...]
    acc = jnp.zeros((m, k), jnp.float32)
    for c in range(32):
        d = z[:, c:c + 1] - embt[c:c + 1, :]
        acc = acc + d * d
    dmin = jnp.min(acc, axis=1, keepdims=True)
    iota = jax.lax.broadcasted_iota(jnp.int32, (m, k), 1)
    lat = jnp.min(jnp.where(acc == dmin, iota, k), axis=1, keepdims=True)
    lat_ref[...] = lat

    onehot = (iota == lat).astype(jnp.float32)
    zq = jnp.dot(onehot, emb_ref[...], preferred_element_type=jnp.float32)
    zq_ref[...] = zq
    hd = jnp.dot(zq, wd_ref[...], preferred_element_type=jnp.float32)
    hd_ref[...] = _packed_bn_relu(hd + bd_ref[...], 1, 16,
                                  gd_ref[...], bbd_ref[...])


def _dec3_body(x_ref, w_ref, b_ref, g_ref, bb_ref, mask_ref, o_ref):
    y = jnp.dot(x_ref[...], w_ref[...], preferred_element_type=jnp.float32)
    y = y + b_ref[...]
    mask = mask_ref[...]
    cnt = 51984.0
    cm = _fold(jnp.sum(y * mask, axis=0, keepdims=True), 32, 16) / (cnt / 32.0)
    yc = y - cm
    cv = _fold(jnp.sum(yc * yc * mask, axis=0, keepdims=True), 32, 16) / (cnt / 32.0)
    o_ref[...] = jnp.maximum(
        g_ref[...] * yc / jnp.sqrt(cv + _EPS) + bb_ref[...], 0.0)


def _dec4_body(x_ref, w_ref, b_ref, g_ref, bb_ref, wo_ref, bo_ref, o_ref):
    y = jnp.dot(x_ref[...], w_ref[...], preferred_element_type=jnp.float32)
    yr = _packed_bn_relu(y + b_ref[...], 32, 16, g_ref[...], bb_ref[...])
    o_ref[...] = (
        jnp.dot(yr, wo_ref[...], preferred_element_type=jnp.float32)
        + bo_ref[...])


def _call(body, out_shapes, *args):
    return pl.pallas_call(body, out_shape=out_shapes)(*args)


def _row(a, reps=1):
    a = jnp.tile(a, reps) if reps > 1 else a
    return a.reshape(1, -1)


def _im2col_s2(h, k, pad):
    n, hh, ww, c = h.shape
    hp = jnp.pad(h, ((0, 0), (pad, pad), (pad, pad), (0, 0)))
    ho = (hh + 2 * pad - k) // 2 + 1
    cols = []
    for kh in range(k):
        for kw in range(k):
            cols.append(hp[:, kh:kh + 2 * ho - 1:2, kw:kw + 2 * ho - 1:2, :])
    x = jnp.concatenate(cols, axis=-1)
    return x.reshape(n * ho * ho, k * k * c), ho


def _wmat(w):
    co, ci, kh, kw = w.shape
    return jnp.transpose(w, (2, 3, 1, 0)).reshape(kh * kw * ci, co)


def _blk(w, reps):
    return jnp.kron(jnp.eye(reps, dtype=w.dtype), w)


def kernel(x, params):
    p = params
    n = x.shape[0]
    xh = jnp.transpose(x, (0, 2, 3, 1))

    x1, h1o = _im2col_s2(xh, 4, 1)
    m1 = x1.shape[0]
    f1 = _call(
        _mm_bn8_body,
        jax.ShapeDtypeStruct((m1 // 8, 128), jnp.float32),
        x1.reshape(m1 // 8, 8 * 48), _blk(_wmat(p['ew1']), 8),
        _row(p['eb1'], 8), _row(p['eg1'], 8), _row(p['ebb1'], 8))
    h1 = f1.reshape(n, h1o, h1o, 16)

    x2, h2o = _im2col_s2(h1, 4, 1)
    m2 = x2.shape[0]
    f2 = _call(
        _mm_bn8_body,
        jax.ShapeDtypeStruct((m2 // 8, 128), jnp.float32),
        x2.reshape(m2 // 8, 8 * 256), _blk(_wmat(p['ew2']), 8),
        _row(p['eb2'], 8), _row(p['eg2'], 8), _row(p['ebb2'], 8))
    h2 = f2.reshape(n, h2o, h2o, 16)

    x3, h3o = _im2col_s2(h2, 2, 1)
    m3 = x3.shape[0]
    w4 = jnp.transpose(p['ew4'][:, :, 0, 0], (1, 0))
    wd1 = p['dw1'][:, :, 0, 0]
    ze_f, lat_f, zq_f, hd1_f = _call(
        _mid_body,
        (jax.ShapeDtypeStruct((m3, 32), jnp.float32),
         jax.ShapeDtypeStruct((m3, 1), jnp.int32),
         jax.ShapeDtypeStruct((m3, 32), jnp.float32),
         jax.ShapeDtypeStruct((m3, 16), jnp.float32)),
        x3, _wmat(p['ew3']), _row(p['eb3']), _row(p['eg3']), _row(p['ebb3']),
        w4, _row(p['eb4']), _row(p['eg4']), _row(p['ebb4']),
        jnp.transpose(p['emb'], (1, 0)), p['emb'],
        wd1, _row(p['db1']), _row(p['dg1']), _row(p['dbb1']))

    z_e_x = jnp.transpose(ze_f.reshape(n, h3o, h3o, 32), (0, 3, 1, 2))
    z_q_x = jnp.transpose(zq_f.reshape(n, h3o, h3o, 32), (0, 3, 1, 2))
    latents = lat_f.reshape(n, h3o, h3o)
    hd1 = hd1_f.reshape(n, h3o, h3o, 16)

    hp = jnp.pad(hd1, ((0, 0), (1, 1), (1, 1), (0, 0)))
    xs, ws = [], []
    w2 = p['dw2']
    for a in range(2):
        for b in range(2):
            taps = []
            wt = []
            for rh in range(2):
                for rw in range(2):
                    taps.append(hp[:, a + rh:a + rh + h3o,
                                   b + rw:b + rw + h3o, :])
                    wt.append(w2[:, :, 3 - a - 2 * rh, 3 - b - 2 * rw])
            xc = jnp.concatenate(taps, axis=-1).reshape(m3 // 4, 4 * 64)
            wc = jnp.concatenate(wt, axis=0)
            xs.append(xc)
            ws.append(_blk(wc, 4))
    xcls = jnp.stack(xs, axis=0)
    wcls = jnp.stack(ws, axis=0)
    d2c = _call(
        _cls4_body,
        jax.ShapeDtypeStruct((4, m3 // 4, 64), jnp.float32),
        xcls, wcls, _row(p['db2'], 4), _row(p['dg2'], 4), _row(p['dbb2'], 4))
    d2 = d2c.reshape(2, 2, n, h3o, h3o, 16)
    d2 = jnp.transpose(d2, (2, 3, 0, 4, 1, 5)).reshape(n, 2 * h3o, 2 * h3o, 16)

    s2b = 2 * h3o
    s3 = s2b - 1
    m5 = n * s2b * s2b
    w3d = p['dw3']
    wall = jnp.concatenate(
        [w3d[:, :, 1 - a, 1 - b] for a in range(2) for b in range(2)], axis=1)
    import numpy as _np
    pos = _np.arange(m5)
    hh = (pos % (s2b * s2b)) // s2b
    wwp = pos % s2b
    vm = []
    for a in range(2):
        for b in range(2):
            vm.append(((hh >= a) & (hh <= s3 - 1 + a)
                       & (wwp >= b) & (wwp <= s3 - 1 + b)))
    mask_np = _np.stack(vm, axis=1).astype(_np.float32)
    mask = jnp.asarray(
        _np.repeat(mask_np, 16, axis=1).reshape(m5 // 8, 512))
    d3c = _call(
        _dec3_body,
        jax.ShapeDtypeStruct((m5 // 8, 512), jnp.float32),
        d2.reshape(m5 // 8, 128), _blk(wall, 8),
        _row(p['db3'], 32), _row(p['dg3'], 32), _row(p['dbb3'], 32), mask)
    d3full = d3c.reshape(n, s2b, s2b, 4, 16)
    cls = []
    for a in range(2):
        for b in range(2):
            cls.append(d3full[:, a:a + s3, b:b + s3, 2 * a + b, :])
    d3 = jnp.stack(cls, axis=0).reshape(2, 2, n, s3, s3, 16)
    d3 = jnp.transpose(d3, (2, 3, 0, 4, 1, 5)).reshape(n, 2 * s3, 2 * s3, 16)

    s4 = 2 * s3
    m6 = n * s4 * s4
    w4d = p['dw4']
    wcat = jnp.concatenate(
        [w4d[:, :, a, b] for a in range(2) for b in range(2)], axis=1)
    wo = p['ow'][:, :, 0, 0]
    zeros = jnp.zeros_like(wo)
    wblk = jnp.concatenate([
        jnp.concatenate([wo if i == j else zeros for j in range(4)], axis=1)
        for i in range(4)], axis=0)
    out6 = _call(
        _dec4_body,
        jax.ShapeDtypeStruct((m6 // 8, 96), jnp.float32),
        d3.reshape(m6 // 8, 128), _blk(wcat, 8),
        _row(p['db4'], 32), _row(p['dg4'], 32), _row(p['dbb4'], 32),
        _blk(wblk, 8), _row(p['ob'], 32))
    xt = out6.reshape(n, s4, s4, 2, 2, 3)
    xt = jnp.transpose(xt, (0, 1, 3, 2, 4, 5)).reshape(n, 2 * s4, 2 * s4, 3)
    x_tilde = jnp.transpose(xt, (0, 3, 1, 2))

    return x_tilde, z_e_x, z_q_x, latents


# speedup vs baseline: 1.6761x; 1.0003x over previous
import jax
import jax.numpy as jnp
from jax.experimental import pallas as pl

_EPS = 1e-5


def _fold(cm, groups, c):
    s = cm[:, 0:c]
    for i in range(1, groups):
        s = s + cm[:, i * c:(i + 1) * c]
    return jnp.concatenate([s / groups] * groups, axis=1)


def _packed_bn_relu(y, groups, c, g, b):
    cm = _fold(jnp.mean(y, axis=0, keepdims=True), groups, c)
    yc = y - cm
    cv = _fold(jnp.mean(yc * yc, axis=0, keepdims=True), groups, c)
    return jnp.maximum(g * yc / jnp.sqrt(cv + _EPS) + b, 0.0)


def _mm_bn8_body(x_ref, w_ref, b_ref, g_ref, bb_ref, o_ref):
    y = jnp.dot(x_ref[...], w_ref[...], preferred_element_type=jnp.float32)
    o_ref[...] = _packed_bn_relu(y + b_ref[...], 8, 16, g_ref[...], bb_ref[...])


def _cls4_body(x_ref, w_ref, b_ref, g_ref, bb_ref, o_ref):
    ys = [
        jnp.dot(x_ref[i], w_ref[i], preferred_element_type=jnp.float32)
        for i in range(4)
    ]
    y = jnp.stack(ys, axis=0) + b_ref[...]
    cm = jnp.mean(y, axis=(0, 1), keepdims=True)
    cm = _fold(cm[0], 4, 16)[None]
    yc = y - cm
    cv = jnp.mean(yc * yc, axis=(0, 1), keepdims=True)
    cv = _fold(cv[0], 4, 16)[None]
    o_ref[...] = jnp.maximum(
        g_ref[...] * yc / jnp.sqrt(cv + _EPS) + bb_ref[...], 0.0)


def _mid_body(x_ref, w3_ref, b3_ref, g3_ref, bb3_ref,
              w4_ref, b4_ref, g4_ref, bb4_ref,
              embt_ref, emb_ref,
              wd_ref, bd_ref, gd_ref, bbd_ref,
              ze_ref, lat_ref, zq_ref, hd_ref):
    h3 = jnp.dot(x_ref[...], w3_ref[...], preferred_element_type=jnp.float32)
    h3 = _packed_bn_relu(h3 + b3_ref[...], 1, 16, g3_ref[...], bb3_ref[...])
    z = jnp.dot(h3, w4_ref[...], preferred_element_type=jnp.float32)
    z = _packed_bn_relu(z + b4_ref[...], 1, 32, g4_ref[...], bb4_ref[...])
    ze_ref[...] = z

    m = z.shape[0]
    k = emb_ref.shape[0]
    embt = embt_ref[...]
    acc = jnp.zeros((m, k), jnp.float32)
    for c in range(32):
        d = z[:, c:c + 1] - embt[c:c + 1, :]
        acc = acc + d * d
    dmin = jnp.min(acc, axis=1, keepdims=True)
    iota = jax.lax.broadcasted_iota(jnp.int32, (m, k), 1)
    lat = jnp.min(jnp.where(acc == dmin, iota, k), axis=1, keepdims=True)
    lat_ref[...] = lat

    onehot = (iota == lat).astype(jnp.float32)
    zq = jnp.dot(onehot, emb_ref[...], preferred_element_type=jnp.float32)
    zq_ref[...] = zq
    hd = jnp.dot(zq, wd_ref[...], preferred_element_type=jnp.float32)
    hd_ref[...] = _packed_bn_relu(hd + bd_ref[...], 1, 16,
                                  gd_ref[...], bbd_ref[...])


def _dec3_body(x_ref, w_ref, b_ref, g_ref, bb_ref, mask_ref, o_ref):
    y = jnp.dot(x_ref[...], w_ref[...], preferred_element_type=jnp.float32)
    y = y + b_ref[...]
    mask = mask_ref[...]
    cnt = 51984.0
    cm = _fold(jnp.sum(y * mask, axis=0, keepdims=True), 32, 16) / (cnt / 32.0)
    yc = y - cm
    cv = _fold(jnp.sum(yc * yc * mask, axis=0, keepdims=True), 32, 16) / (cnt / 32.0)
    o_ref[...] = jnp.maximum(
        g_ref[...] * yc / jnp.sqrt(cv + _EPS) + bb_ref[...], 0.0)


def _dec4_body(x_ref, w_ref, b_ref, g_ref, bb_ref, wo_ref, bo_ref, o_ref):
    y = jnp.dot(x_ref[...], w_ref[...], preferred_element_type=jnp.float32)
    yr = _packed_bn_relu(y + b_ref[...], 32, 16, g_ref[...], bb_ref[...])
    o_ref[...] = (
        jnp.dot(yr, wo_ref[...], preferred_element_type=jnp.float32)
        + bo_ref[...])


def _call(body, out_shapes, *args):
    return pl.pallas_call(body, out_shape=out_shapes)(*args)


def _row(a, reps=1):
    a = jnp.tile(a, reps) if reps > 1 else a
    return a.reshape(1, -1)


def _im2col_s2(h, k, pad):
    n, hh, ww, c = h.shape
    hp = jnp.pad(h, ((0, 0), (pad, pad), (pad, pad), (0, 0)))
    ho = (hh + 2 * pad - k) // 2 + 1
    cols = []
    for kh in range(k):
        for kw in range(k):
            cols.append(hp[:, kh:kh + 2 * ho - 1:2, kw:kw + 2 * ho - 1:2, :])
    x = jnp.concatenate(cols, axis=-1)
    return x.reshape(n * ho * ho, k * k * c), ho


def _wmat(w):
    co, ci, kh, kw = w.shape
    return jnp.transpose(w, (2, 3, 1, 0)).reshape(kh * kw * ci, co)


def _blk(w, reps):
    return jnp.kron(jnp.eye(reps, dtype=w.dtype), w)


def kernel(x, params):
    p = params
    n = x.shape[0]
    xh = jax.lax.optimization_barrier(jnp.transpose(x, (0, 2, 3, 1)))

    x1, h1o = _im2col_s2(xh, 4, 1)
    m1 = x1.shape[0]
    f1 = _call(
        _mm_bn8_body,
        jax.ShapeDtypeStruct((m1 // 8, 128), jnp.float32),
        x1.reshape(m1 // 8, 8 * 48), _blk(_wmat(p['ew1']), 8),
        _row(p['eb1'], 8), _row(p['eg1'], 8), _row(p['ebb1'], 8))
    h1 = f1.reshape(n, h1o, h1o, 16)

    x2, h2o = _im2col_s2(h1, 4, 1)
    m2 = x2.shape[0]
    f2 = _call(
        _mm_bn8_body,
        jax.ShapeDtypeStruct((m2 // 8, 128), jnp.float32),
        x2.reshape(m2 // 8, 8 * 256), _blk(_wmat(p['ew2']), 8),
        _row(p['eb2'], 8), _row(p['eg2'], 8), _row(p['ebb2'], 8))
    h2 = f2.reshape(n, h2o, h2o, 16)

    x3, h3o = _im2col_s2(h2, 2, 1)
    m3 = x3.shape[0]
    w4 = jnp.transpose(p['ew4'][:, :, 0, 0], (1, 0))
    wd1 = p['dw1'][:, :, 0, 0]
    ze_f, lat_f, zq_f, hd1_f = _call(
        _mid_body,
        (jax.ShapeDtypeStruct((m3, 32), jnp.float32),
         jax.ShapeDtypeStruct((m3, 1), jnp.int32),
         jax.ShapeDtypeStruct((m3, 32), jnp.float32),
         jax.ShapeDtypeStruct((m3, 16), jnp.float32)),
        x3, _wmat(p['ew3']), _row(p['eb3']), _row(p['eg3']), _row(p['ebb3']),
        w4, _row(p['eb4']), _row(p['eg4']), _row(p['ebb4']),
        jnp.transpose(p['emb'], (1, 0)), p['emb'],
        wd1, _row(p['db1']), _row(p['dg1']), _row(p['dbb1']))

    z_e_x = jnp.transpose(ze_f.reshape(n, h3o, h3o, 32), (0, 3, 1, 2))
    z_q_x = jnp.transpose(zq_f.reshape(n, h3o, h3o, 32), (0, 3, 1, 2))
    latents = lat_f.reshape(n, h3o, h3o)
    hd1 = hd1_f.reshape(n, h3o, h3o, 16)

    hp = jnp.pad(hd1, ((0, 0), (1, 1), (1, 1), (0, 0)))
    xs, ws = [], []
    w2 = p['dw2']
    for a in range(2):
        for b in range(2):
            taps = []
            wt = []
            for rh in range(2):
                for rw in range(2):
                    taps.append(hp[:, a + rh:a + rh + h3o,
                                   b + rw:b + rw + h3o, :])
                    wt.append(w2[:, :, 3 - a - 2 * rh, 3 - b - 2 * rw])
            xc = jnp.concatenate(taps, axis=-1).reshape(m3 // 4, 4 * 64)
            wc = jnp.concatenate(wt, axis=0)
            xs.append(xc)
            ws.append(_blk(wc, 4))
    xcls = jnp.stack(xs, axis=0)
    wcls = jnp.stack(ws, axis=0)
    d2c = _call(
        _cls4_body,
        jax.ShapeDtypeStruct((4, m3 // 4, 64), jnp.float32),
        xcls, wcls, _row(p['db2'], 4), _row(p['dg2'], 4), _row(p['dbb2'], 4))
    d2 = d2c.reshape(2, 2, n, h3o, h3o, 16)
    d2 = jnp.transpose(d2, (2, 3, 0, 4, 1, 5)).reshape(n, 2 * h3o, 2 * h3o, 16)

    s2b = 2 * h3o
    s3 = s2b - 1
    m5 = n * s2b * s2b
    w3d = p['dw3']
    wall = jnp.concatenate(
        [w3d[:, :, 1 - a, 1 - b] for a in range(2) for b in range(2)], axis=1)
    import numpy as _np
    pos = _np.arange(m5)
    hh = (pos % (s2b * s2b)) // s2b
    wwp = pos % s2b
    vm = []
    for a in range(2):
        for b in range(2):
            vm.append(((hh >= a) & (hh <= s3 - 1 + a)
                       & (wwp >= b) & (wwp <= s3 - 1 + b)))
    mask_np = _np.stack(vm, axis=1).astype(_np.float32)
    mask = jnp.asarray(
        _np.repeat(mask_np, 16, axis=1).reshape(m5 // 8, 512))
    d3c = _call(
        _dec3_body,
        jax.ShapeDtypeStruct((m5 // 8, 512), jnp.float32),
        d2.reshape(m5 // 8, 128), _blk(wall, 8),
        _row(p['db3'], 32), _row(p['dg3'], 32), _row(p['dbb3'], 32), mask)
    d3full = d3c.reshape(n, s2b, s2b, 4, 16)
    cls = []
    for a in range(2):
        for b in range(2):
            cls.append(d3full[:, a:a + s3, b:b + s3, 2 * a + b, :])
    d3 = jnp.stack(cls, axis=0).reshape(2, 2, n, s3, s3, 16)
    d3 = jnp.transpose(d3, (2, 3, 0, 4, 1, 5)).reshape(n, 2 * s3, 2 * s3, 16)

    s4 = 2 * s3
    m6 = n * s4 * s4
    w4d = p['dw4']
    wcat = jnp.concatenate(
        [w4d[:, :, a, b] for a in range(2) for b in range(2)], axis=1)
    wo = p['ow'][:, :, 0, 0]
    zeros = jnp.zeros_like(wo)
    wblk = jnp.concatenate([
        jnp.concatenate([wo if i == j else zeros for j in range(4)], axis=1)
        for i in range(4)], axis=0)
    out6 = _call(
        _dec4_body,
        jax.ShapeDtypeStruct((m6 // 8, 96), jnp.float32),
        d3.reshape(m6 // 8, 128), _blk(wcat, 8),
        _row(p['db4'], 32), _row(p['dg4'], 32), _row(p['dbb4'], 32),
        _blk(wblk, 8), _row(p['ob'], 32))
    xt = out6.reshape(n, s4, s4, 2, 2, 3)
    xt = jnp.transpose(xt, (0, 1, 3, 2, 4, 5)).reshape(n, 2 * s4, 2 * s4, 3)
    x_tilde = jnp.transpose(xt, (0, 3, 1, 2))

    return x_tilde, z_e_x, z_q_x, latents


# speedup vs baseline: 4.0413x; 2.4111x over previous
import jax
import jax.numpy as jnp
from jax.experimental import pallas as pl

_EPS = 1e-5


def _bn_relu_cm(y, g, b):
    m = jnp.mean(y, axis=(0, 2), keepdims=True)
    v = jnp.mean((y - m) ** 2, axis=(0, 2), keepdims=True)
    return jnp.maximum(g * (y - m) / jnp.sqrt(v + _EPS) + b, 0.0)


def _enc_body(x_ref, w_ref, b_ref, g_ref, bb_ref, o_ref):
    n, k, hh, ww = x_ref.shape
    x = x_ref[...].reshape(n, k, hh * ww)
    w = w_ref[...]
    y = jnp.stack(
        [jnp.dot(w.T, x[i], preferred_element_type=jnp.float32)
         for i in range(n)], axis=0)
    y = _bn_relu_cm(y + b_ref[...], g_ref[...], bb_ref[...])
    o_ref[...] = y.reshape(n, 16, hh, ww)


def _mid_body(x_ref, w3_ref, b3_ref, g3_ref, bb3_ref,
              w4_ref, b4_ref, g4_ref, bb4_ref,
              emb_ref, embt_ref,
              wd_ref, bd_ref, gd_ref, bbd_ref,
              ze_ref, lat_ref, zq_ref, hd_ref):
    n = x_ref.shape[0]
    x = x_ref[...].reshape(n, 64, 841)
    h3 = jnp.stack(
        [jnp.dot(w3_ref[...].T, x[i], preferred_element_type=jnp.float32)
         for i in range(n)], axis=0)
    h3 = _bn_relu_cm(h3 + b3_ref[...], g3_ref[...], bb3_ref[...])
    z = jnp.stack(
        [jnp.dot(w4_ref[...], h3[i], preferred_element_type=jnp.float32)
         for i in range(n)], axis=0)
    z = _bn_relu_cm(z + b4_ref[...], g4_ref[...], bb4_ref[...])
    ze_ref[...] = z.reshape(n, 32, 29, 29)

    emb = emb_ref[...]
    embt = embt_ref[...]
    lats, zqs = [], []
    it = jax.lax.broadcasted_iota(jnp.int32, (512, 841), 0)
    for i in range(n):
        acc = jnp.zeros((512, 841), jnp.float32)
        for c in range(32):
            d = z[i, c:c + 1, :] - emb[:, c:c + 1]
            acc = acc + d * d
        mn = jnp.min(acc, axis=0, keepdims=True)
        lat = jnp.min(jnp.where(acc == mn, it, 512), axis=0,
                      keepdims=True)
        lats.append(lat)
        onehot = (it == lat).astype(jnp.float32)
        zqs.append(jnp.dot(embt, onehot, preferred_element_type=jnp.float32))
    lat_ref[...] = jnp.concatenate(lats, axis=0)
    zq = jnp.stack(zqs, axis=0)
    zq_ref[...] = zq.reshape(n, 32, 29, 29)

    hd = jnp.stack(
        [jnp.dot(wd_ref[...].T, zq[i], preferred_element_type=jnp.float32)
         for i in range(n)], axis=0)
    hd = _bn_relu_cm(hd + bd_ref[...], gd_ref[...], bbd_ref[...])
    hd = hd.reshape(n, 16, 29, 29)
    zc = jnp.zeros((n, 16, 29, 1), jnp.float32)
    hd = jnp.concatenate([zc, hd, zc], axis=3)
    zr = jnp.zeros((n, 16, 1, 31), jnp.float32)
    hd_ref[...] = jnp.concatenate([zr, hd, zr], axis=2)


def _ilv(p00, p01, p10, p11):
    n, c, h, w = p00.shape
    q0 = jnp.stack([p00, p01], axis=-1).reshape(n, c, h, 2 * w)
    q1 = jnp.stack([p10, p11], axis=-1).reshape(n, c, h, 2 * w)
    return jnp.stack([q0, q1], axis=3).reshape(n, c, 2 * h, 2 * w)


def _deccls_body(x_ref, w_ref, b_ref, g_ref, bb_ref, o_ref):
    n, kc, hh, ww = x_ref.shape
    k = kc // 4
    x = x_ref[...].reshape(n, kc, hh * ww)
    w = w_ref[...]
    ys = []
    for cls in range(4):
        wc = w[cls]
        ys.append(jnp.stack(
            [jnp.dot(wc.T, x[i, cls * k:(cls + 1) * k, :],
                     preferred_element_type=jnp.float32)
             for i in range(n)], axis=0))
    y = jnp.stack(ys, axis=0) + b_ref[...]
    m = jnp.mean(y, axis=(0, 1, 3), keepdims=True)
    v = jnp.mean((y - m) ** 2, axis=(0, 1, 3), keepdims=True)
    o_ref[...] = jnp.maximum(
        g_ref[...] * (y - m) / jnp.sqrt(v + _EPS) + bb_ref[...], 0.0)


def _dec4_body(x_ref, w4_ref, b4_ref, g4_ref, bb4_ref, wo_ref, bo_ref,
               o_ref):
    n = 4
    y3 = x_ref[...]
    w4 = w4_ref[...]
    ys = []
    for c3 in range(4):
        yc = y3[c3]
        ys.append(jnp.stack(
            [jnp.dot(w4.T, yc[i], preferred_element_type=jnp.float32)
             for i in range(n)], axis=0))
    y = jnp.stack(ys, axis=0) + b4_ref[...]
    cm = jnp.mean(y, axis=(0, 1, 3), keepdims=True)
    m16 = (cm[:, :, 0:16] + cm[:, :, 16:32]
           + cm[:, :, 32:48] + cm[:, :, 48:64]) * 0.25
    mt = jnp.concatenate([m16] * 4, axis=2)
    yc = y - mt
    cv = jnp.mean(yc * yc, axis=(0, 1, 3), keepdims=True)
    v16 = (cv[:, :, 0:16] + cv[:, :, 16:32]
           + cv[:, :, 32:48] + cv[:, :, 48:64]) * 0.25
    vt = jnp.concatenate([v16] * 4, axis=2)
    yr = jnp.maximum(
        g4_ref[...] * yc / jnp.sqrt(vt + _EPS) + bb4_ref[...], 0.0)
    wo = wo_ref[...]
    outs = []
    for c3 in range(4):
        yrc = yr[c3]
        outs.append(jnp.stack(
            [jnp.dot(wo.T, yrc[i], preferred_element_type=jnp.float32)
             for i in range(n)], axis=0))
    o_ref[...] = jnp.stack(outs, axis=0) + bo_ref[...]


def _asm_body(p_ref, o_ref):
    p = p_ref[...]
    for co in range(3):
        ms = []
        for r in range(4):
            a3, a4 = r // 2, r % 2
            cols = [p[:, 2 * a3 + b3, 0, 3 * (2 * a4 + b4) + co, :]
                    for b3 in range(2) for b4 in range(2)]
            ms.append(jnp.stack(cols, axis=-1).reshape(57, 228))
        o_ref[0, co] = jnp.stack(ms, axis=1).reshape(228, 228)


def _taps_s2(xp, k, ho):
    cols = []
    for kh in range(k):
        for kw in range(k):
            cols.append(xp[:, :, kh:kh + 2 * ho - 1:2, kw:kw + 2 * ho - 1:2])
    return jnp.concatenate(cols, axis=1)


def _wmat_cm(w):
    co, ci, kh, kw = w.shape
    return jnp.transpose(w, (2, 3, 1, 0)).reshape(kh * kw * ci, co)


def _c3(a):
    return a.reshape(1, -1, 1)


def kernel(x, params):
    p = params
    n = x.shape[0]

    xp = jnp.pad(x, ((0, 0), (0, 0), (1, 1), (1, 1)))
    x1 = _taps_s2(xp, 4, 112)
    h1 = pl.pallas_call(
        _enc_body,
        out_shape=jax.ShapeDtypeStruct((n, 16, 112, 112), jnp.float32),
    )(x1, _wmat_cm(p['ew1']), _c3(p['eb1']), _c3(p['eg1']), _c3(p['ebb1']))

    h1p = jnp.pad(h1, ((0, 0), (0, 0), (1, 1), (1, 1)))
    x2 = _taps_s2(h1p, 4, 56)
    h2 = pl.pallas_call(
        _enc_body,
        out_shape=jax.ShapeDtypeStruct((n, 16, 56, 56), jnp.float32),
    )(x2, _wmat_cm(p['ew2']), _c3(p['eb2']), _c3(p['eg2']), _c3(p['ebb2']))

    h2p = jnp.pad(h2, ((0, 0), (0, 0), (1, 1), (1, 1)))
    x3 = _taps_s2(h2p, 2, 29)

    w4 = jnp.transpose(p['ew4'][:, :, 0, 0], (1, 0)).T
    wd1 = p['dw1'][:, :, 0, 0]
    ze, lat, zq, hdp = pl.pallas_call(
        _mid_body,
        out_shape=(jax.ShapeDtypeStruct((n, 32, 29, 29), jnp.float32),
                   jax.ShapeDtypeStruct((n, 841), jnp.int32),
                   jax.ShapeDtypeStruct((n, 32, 29, 29), jnp.float32),
                   jax.ShapeDtypeStruct((n, 16, 31, 31), jnp.float32)),
    )(x3, _wmat_cm(p['ew3']), _c3(p['eb3']), _c3(p['eg3']), _c3(p['ebb3']),
      w4, _c3(p['eb4']), _c3(p['eg4']), _c3(p['ebb4']),
      p['emb'], jnp.transpose(p['emb'], (1, 0)),
      wd1, _c3(p['db1']), _c3(p['dg1']), _c3(p['dbb1']))

    latents = lat.reshape(n, 29, 29)

    w2t = p['dw2']
    w2c = jnp.stack([
        jnp.concatenate(
            [w2t[:, :, 3 - a - 2 * rh, 3 - b - 2 * rw]
             for rh in range(2) for rw in range(2)], axis=0)
        for a in range(2) for b in range(2)], axis=0)
    w3c = jnp.stack([p['dw3'][:, :, 1 - a, 1 - b]
                     for a in range(2) for b in range(2)], axis=0)
    w4cat = jnp.concatenate(
        [p['dw4'][:, :, a, b] for a in range(2) for b in range(2)],
        axis=1)
    wo = p['ow'][:, :, 0, 0]
    zo = jnp.zeros_like(wo)
    wo_blk = jnp.concatenate([
        jnp.concatenate([wo if i == j else zo for j in range(4)], axis=1)
        for i in range(4)], axis=0)

    def c4(a, reps=1):
        a = jnp.tile(a, reps) if reps > 1 else a
        return a.reshape(1, 1, -1, 1)

    x2d = jnp.concatenate(
        [hdp[:, :, a + rh:a + rh + 29, b + rw:b + rw + 29]
         for a in range(2) for b in range(2)
         for rh in range(2) for rw in range(2)], axis=1)
    y2 = pl.pallas_call(
        _deccls_body,
        out_shape=jax.ShapeDtypeStruct((4, n, 16, 841), jnp.float32),
    )(x2d, w2c, c4(p['db2']), c4(p['dg2']), c4(p['dbb2']))
    d2 = jnp.transpose(
        y2.reshape(2, 2, n, 16, 29, 29),
        (2, 3, 4, 0, 5, 1)).reshape(n, 16, 58, 58)

    x3d = jnp.concatenate(
        [d2[:, :, a:a + 57, b:b + 57]
         for a in range(2) for b in range(2)], axis=1)
    y3 = pl.pallas_call(
        _deccls_body,
        out_shape=jax.ShapeDtypeStruct((4, n, 16, 3249), jnp.float32),
    )(x3d, w3c, c4(p['db3']), c4(p['dg3']), c4(p['dbb3']))

    ot = pl.pallas_call(
        _dec4_body,
        out_shape=jax.ShapeDtypeStruct((4, n, 12, 3249), jnp.float32),
    )(y3, w4cat, c4(p['db4'], 4), c4(p['dg4'], 4), c4(p['dbb4'], 4),
      wo_blk, c4(p['ob'], 4))

    ot5 = jnp.transpose(ot.reshape(4, n, 12, 57, 57), (3, 0, 1, 2, 4))
    x_tilde = pl.pallas_call(
        _asm_body,
        grid=(n,),
        in_specs=[pl.BlockSpec((57, 4, 1, 12, 57),
                               lambda i: (0, 0, i, 0, 0))],
        out_specs=pl.BlockSpec((1, 3, 228, 228), lambda i: (i, 0, 0, 0)),
        out_shape=jax.ShapeDtypeStruct((n, 3, 228, 228), jnp.float32),
    )(ot5)

    return x_tilde, ze, zq, latents
